# baseline (speedup 1.0000x reference)
import functools

import jax
import jax.numpy as jnp
from jax import lax
from jax.experimental import pallas as pl
from jax.experimental.pallas import tpu as pltpu
from jax.experimental.pallas import tpu_sc as plsc

N = 10000
E = 320000
D = 128

NC = 2
NS = 16
NW = NC * NS

NPAD = 10240
CH = 64
NCHUNK = 160
EPW = CH * NCHUNK
EPAD = NW * EPW
RPS = NPAD // NS
DUMP = NPAD - 1
CCH = 128
CNCHUNK = EPW // CCH


IB = 40
NBLK = NCHUNK // IB


def _sc_agg_body(h_hbm, src3_hbm, dst3_hbm, z128_hbm, agg_out,
                 sidx_blk, didx_blk, rows0, rows1,
                 gsem0, gsem1, ssem0, ssem1, acc):
  c = lax.axis_index("c")
  s = lax.axis_index("s")
  wid = c * NS + s
  r0 = s * RPS
  rows = (rows0, rows1)
  gsem = (gsem0, gsem1)
  ssem = (ssem0, ssem1)

  pltpu.sync_copy(z128_hbm.at[pl.ds(r0, RPS), :], acc.at[pl.ds(r0, RPS), :])
  plsc.subcore_barrier()

  def gather_start(j, b):
    pltpu.async_copy(h_hbm.at[sidx_blk.at[j]], rows[b], gsem[b])

  def gather_wait(j, b):
    pltpu.make_async_copy(h_hbm.at[sidx_blk.at[j]], rows[b], gsem[b]).wait()

  def scat_start(j, b):
    pltpu.async_copy(rows[b], acc.at[didx_blk.at[j]], ssem[b], add=True)

  def scat_wait(j, b):
    pltpu.make_async_copy(rows[b], acc.at[didx_blk.at[j]], ssem[b]).wait()

  @pl.loop(0, NBLK)
  def _(blk):
    pltpu.sync_copy(src3_hbm.at[wid, pl.ds(blk * IB, IB)], sidx_blk)
    pltpu.sync_copy(dst3_hbm.at[wid, pl.ds(blk * IB, IB)], didx_blk)
    gather_start(0, 0)

    @pl.loop(0, IB // 2)
    def _(p):
      for b in range(2):
        j = 2 * p + b
        gather_wait(j, b)
        scat_start(j, b)

        @pl.when(j >= 1)
        def _():
          scat_wait(j - 1, 1 - b)

        @pl.when(j + 1 < IB)
        def _():
          gather_start(j + 1, 1 - b)

    scat_wait(IB - 1, 1)

  plsc.subcore_barrier()
  pltpu.sync_copy(acc.at[pl.ds(r0, RPS), :], agg_out.at[c, pl.ds(r0, RPS), :])


_sc_agg = pl.kernel(
    _sc_agg_body,
    out_type=jax.ShapeDtypeStruct((NC, NPAD, D), jnp.float32),
    mesh=plsc.VectorSubcoreMesh(core_axis_name="c", subcore_axis_name="s",
                                num_cores=NC, num_subcores=NS),
    scratch_types=[
        pltpu.VMEM((IB, CH), jnp.int32),
        pltpu.VMEM((IB, CH), jnp.int32),
        pltpu.VMEM((CH, D), jnp.float32),
        pltpu.VMEM((CH, D), jnp.float32),
        pltpu.SemaphoreType.DMA,
        pltpu.SemaphoreType.DMA,
        pltpu.SemaphoreType.DMA,
        pltpu.SemaphoreType.DMA,
        pltpu.VMEM_SHARED((NPAD, D), jnp.float32),
    ],
)


def _sc_count_body(dst3_hbm, ones_hbm, z128_hbm, cnt_out, didx_all, ones,
                   ssem, acc):
  c = lax.axis_index("c")
  s = lax.axis_index("s")
  wid = c * NS + s
  r0 = s * RPS

  pltpu.sync_copy(dst3_hbm.at[wid], didx_all)
  pltpu.sync_copy(z128_hbm.at[pl.ds(r0, RPS), :], acc.at[pl.ds(r0, RPS), :])
  pltpu.sync_copy(ones_hbm, ones)
  plsc.subcore_barrier()

  @pl.loop(0, NCHUNK)
  def _(i):
    pltpu.async_copy(ones, acc.at[didx_all.at[i]], ssem, add=True)

    @pl.when(i > 0)
    def _():
      pltpu.make_async_copy(ones, acc.at[didx_all.at[i]], ssem).wait()

  pltpu.make_async_copy(ones, acc.at[didx_all.at[0]], ssem).wait()
  plsc.subcore_barrier()
  pltpu.sync_copy(acc.at[pl.ds(r0, RPS), :], cnt_out.at[c, pl.ds(r0, RPS), :])


_sc_count = pl.kernel(
    _sc_count_body,
    out_type=jax.ShapeDtypeStruct((NC, NPAD, D), jnp.float32),
    mesh=plsc.VectorSubcoreMesh(core_axis_name="c", subcore_axis_name="s",
                                num_cores=NC, num_subcores=NS),
    scratch_types=[
        pltpu.VMEM((NCHUNK, CH), jnp.int32),
        pltpu.VMEM((CH, D), jnp.float32),
        pltpu.SemaphoreType.DMA,
        pltpu.VMEM_SHARED((NPAD, D), jnp.float32),
    ],
)

RB = 2048


def _tc_layer_body(relu, agg_ref, cnt_ref, h_ref, wl_ref, bl_ref, wr_ref,
                   out_ref):
  a = agg_ref[0] + agg_ref[1]
  cnt = cnt_ref[0, :, :8] + cnt_ref[1, :, :8]
  inv = 1.0 / jnp.maximum(cnt[:, :1], 1.0)
  m = a * inv
  dn = (((1,), (1,)), ((), ()))
  y = (lax.dot_general(m, wl_ref[...], dn, preferred_element_type=jnp.float32)
       + lax.dot_general(h_ref[...], wr_ref[...], dn,
                         preferred_element_type=jnp.float32)
       + bl_ref[...])
  out_ref[...] = jnp.maximum(y, 0.0) if relu else y


def _tc_layer(relu, agg2, cnt2, h, Wl, bl, Wr):
  return pl.pallas_call(
      functools.partial(_tc_layer_body, relu),
      grid=(NPAD // RB,),
      in_specs=[
          pl.BlockSpec((NC, RB, D), lambda i: (0, i, 0)),
          pl.BlockSpec((NC, RB, D), lambda i: (0, i, 0)),
          pl.BlockSpec((RB, D), lambda i: (i, 0)),
          pl.BlockSpec((D, D), lambda i: (0, 0)),
          pl.BlockSpec((1, D), lambda i: (0, 0)),
          pl.BlockSpec((D, D), lambda i: (0, 0)),
      ],
      out_specs=pl.BlockSpec((RB, D), lambda i: (i, 0)),
      out_shape=jax.ShapeDtypeStruct((NPAD, D), jnp.float32),
  )(agg2, cnt2, h, Wl, bl.reshape(1, D), Wr)


def kernel(x, edge_index, Wl0, bl0, Wr0, Wl1, bl1, Wr1, Wl2, bl2, Wr2):
  epad = EPAD - E
  src = jnp.concatenate(
      [edge_index[0].astype(jnp.int32),
       jnp.arange(epad, dtype=jnp.int32) % N])
  dst = jnp.concatenate(
      [edge_index[1].astype(jnp.int32), jnp.full((epad,), DUMP, jnp.int32)])
  src = src.reshape(NW, NCHUNK, CH)
  dst = dst.reshape(NW, NCHUNK, CH)
  h = jnp.pad(x, ((0, NPAD - N), (0, 0)))
  z128 = jnp.zeros((NPAD, D), jnp.float32)
  ones128 = jnp.ones((CH, D), jnp.float32)

  cnt2 = _sc_count(dst, ones128, z128)
  agg2 = _sc_agg(h, src, dst, z128)
  h = _tc_layer(True, agg2, cnt2, h, Wl0, bl0, Wr0)
  agg2 = _sc_agg(h, src, dst, z128)
  h = _tc_layer(True, agg2, cnt2, h, Wl1, bl1, Wr1)
  agg2 = _sc_agg(h, src, dst, z128)
  return _tc_layer(False, agg2, cnt2, h, Wl2, bl2, Wr2)[:N]


# speedup vs baseline: 1.2888x; 1.2888x over previous
import functools

import jax
import jax.numpy as jnp
from jax import lax
from jax.experimental import pallas as pl
from jax.experimental.pallas import tpu as pltpu
from jax.experimental.pallas import tpu_sc as plsc

N = 10000
E = 320000
D = 128

NC = 2
NS = 16
NW = NC * NS

NPAD = 10240
CH = 128
NCHUNK = 80
EPW = CH * NCHUNK
EPAD = NW * EPW
RPS = NPAD // NS
DUMP = NPAD - 1
CCH = 128
CNCHUNK = EPW // CCH


IB = 40
NBLK = NCHUNK // IB


def _sc_agg_body(h_hbm, src3_hbm, dst3_hbm, z128_hbm, agg_out,
                 sidx_blk, didx_blk, rows0, rows1,
                 gsem0, gsem1, ssem0, ssem1, acc):
  c = lax.axis_index("c")
  s = lax.axis_index("s")
  wid = c * NS + s
  r0 = s * RPS
  rows = (rows0, rows1)
  gsem = (gsem0, gsem1)
  ssem = (ssem0, ssem1)

  pltpu.sync_copy(z128_hbm.at[pl.ds(r0, RPS), :], acc.at[pl.ds(r0, RPS), :])
  plsc.subcore_barrier()

  def gather_start(j, b):
    pltpu.async_copy(h_hbm.at[sidx_blk.at[j]], rows[b], gsem[b])

  def gather_wait(j, b):
    pltpu.make_async_copy(h_hbm.at[sidx_blk.at[j]], rows[b], gsem[b]).wait()

  def scat_start(j, b):
    pltpu.async_copy(rows[b], acc.at[didx_blk.at[j]], ssem[b], add=True)

  def scat_wait(j, b):
    pltpu.make_async_copy(rows[b], acc.at[didx_blk.at[j]], ssem[b]).wait()

  @pl.loop(0, NBLK)
  def _(blk):
    pltpu.sync_copy(src3_hbm.at[wid, pl.ds(blk * IB, IB)], sidx_blk)
    pltpu.sync_copy(dst3_hbm.at[wid, pl.ds(blk * IB, IB)], didx_blk)
    gather_start(0, 0)

    @pl.loop(0, IB // 2)
    def _(p):
      for b in range(2):
        j = 2 * p + b
        gather_wait(j, b)
        scat_start(j, b)

        @pl.when(j >= 1)
        def _():
          scat_wait(j - 1, 1 - b)

        @pl.when(j + 1 < IB)
        def _():
          gather_start(j + 1, 1 - b)

    scat_wait(IB - 1, 1)

  plsc.subcore_barrier()
  pltpu.sync_copy(acc.at[pl.ds(r0, RPS), :], agg_out.at[c, pl.ds(r0, RPS), :])


_sc_agg = pl.kernel(
    _sc_agg_body,
    out_type=jax.ShapeDtypeStruct((NC, NPAD, D), jnp.float32),
    mesh=plsc.VectorSubcoreMesh(core_axis_name="c", subcore_axis_name="s",
                                num_cores=NC, num_subcores=NS),
    scratch_types=[
        pltpu.VMEM((IB, CH), jnp.int32),
        pltpu.VMEM((IB, CH), jnp.int32),
        pltpu.VMEM((CH, D), jnp.float32),
        pltpu.VMEM((CH, D), jnp.float32),
        pltpu.SemaphoreType.DMA,
        pltpu.SemaphoreType.DMA,
        pltpu.SemaphoreType.DMA,
        pltpu.SemaphoreType.DMA,
        pltpu.VMEM_SHARED((NPAD, D), jnp.float32),
    ],
)


def _sc_count_body(dst3_hbm, ones_hbm, z128_hbm, cnt_out, didx_all, ones,
                   ssem, acc):
  c = lax.axis_index("c")
  s = lax.axis_index("s")
  wid = c * NS + s
  r0 = s * RPS

  pltpu.sync_copy(dst3_hbm.at[wid], didx_all)
  pltpu.sync_copy(z128_hbm.at[pl.ds(r0, RPS), :], acc.at[pl.ds(r0, RPS), :])
  pltpu.sync_copy(ones_hbm, ones)
  plsc.subcore_barrier()

  @pl.loop(0, NCHUNK)
  def _(i):
    pltpu.async_copy(ones, acc.at[didx_all.at[i]], ssem, add=True)

    @pl.when(i > 0)
    def _():
      pltpu.make_async_copy(ones, acc.at[didx_all.at[i]], ssem).wait()

  pltpu.make_async_copy(ones, acc.at[didx_all.at[0]], ssem).wait()
  plsc.subcore_barrier()
  pltpu.sync_copy(acc.at[pl.ds(r0, RPS), :], cnt_out.at[c, pl.ds(r0, RPS), :])


_sc_count = pl.kernel(
    _sc_count_body,
    out_type=jax.ShapeDtypeStruct((NC, NPAD, D), jnp.float32),
    mesh=plsc.VectorSubcoreMesh(core_axis_name="c", subcore_axis_name="s",
                                num_cores=NC, num_subcores=NS),
    scratch_types=[
        pltpu.VMEM((NCHUNK, CH), jnp.int32),
        pltpu.VMEM((CH, D), jnp.float32),
        pltpu.SemaphoreType.DMA,
        pltpu.VMEM_SHARED((NPAD, D), jnp.float32),
    ],
)

RB = 2048


def _tc_layer_body(relu, agg_ref, cnt_ref, h_ref, wl_ref, bl_ref, wr_ref,
                   out_ref):
  a = agg_ref[0] + agg_ref[1]
  cnt = cnt_ref[0, :, :8] + cnt_ref[1, :, :8]
  inv = 1.0 / jnp.maximum(cnt[:, :1], 1.0)
  m = a * inv
  dn = (((1,), (1,)), ((), ()))
  y = (lax.dot_general(m, wl_ref[...], dn, preferred_element_type=jnp.float32)
       + lax.dot_general(h_ref[...], wr_ref[...], dn,
                         preferred_element_type=jnp.float32)
       + bl_ref[...])
  out_ref[...] = jnp.maximum(y, 0.0) if relu else y


def _tc_layer(relu, agg2, cnt2, h, Wl, bl, Wr):
  return pl.pallas_call(
      functools.partial(_tc_layer_body, relu),
      grid=(NPAD // RB,),
      in_specs=[
          pl.BlockSpec((NC, RB, D), lambda i: (0, i, 0)),
          pl.BlockSpec((NC, RB, D), lambda i: (0, i, 0)),
          pl.BlockSpec((RB, D), lambda i: (i, 0)),
          pl.BlockSpec((D, D), lambda i: (0, 0)),
          pl.BlockSpec((1, D), lambda i: (0, 0)),
          pl.BlockSpec((D, D), lambda i: (0, 0)),
      ],
      out_specs=pl.BlockSpec((RB, D), lambda i: (i, 0)),
      out_shape=jax.ShapeDtypeStruct((NPAD, D), jnp.float32),
  )(agg2, cnt2, h, Wl, bl.reshape(1, D), Wr)


def kernel(x, edge_index, Wl0, bl0, Wr0, Wl1, bl1, Wr1, Wl2, bl2, Wr2):
  epad = EPAD - E
  src = jnp.concatenate(
      [edge_index[0].astype(jnp.int32),
       jnp.arange(epad, dtype=jnp.int32) % N])
  dst = jnp.concatenate(
      [edge_index[1].astype(jnp.int32), jnp.full((epad,), DUMP, jnp.int32)])
  src = src.reshape(NW, NCHUNK, CH)
  dst = dst.reshape(NW, NCHUNK, CH)
  h = jnp.pad(x, ((0, NPAD - N), (0, 0)))
  z128 = jnp.zeros((NPAD, D), jnp.float32)
  ones128 = jnp.ones((CH, D), jnp.float32)

  cnt2 = _sc_count(dst, ones128, z128)
  agg2 = _sc_agg(h, src, dst, z128)
  h = _tc_layer(True, agg2, cnt2, h, Wl0, bl0, Wr0)
  agg2 = _sc_agg(h, src, dst, z128)
  h = _tc_layer(True, agg2, cnt2, h, Wl1, bl1, Wr1)
  agg2 = _sc_agg(h, src, dst, z128)
  return _tc_layer(False, agg2, cnt2, h, Wl2, bl2, Wr2)[:N]


# speedup vs baseline: 1.2926x; 1.0030x over previous
import functools

import jax
import jax.numpy as jnp
from jax import lax
from jax.experimental import pallas as pl
from jax.experimental.pallas import tpu as pltpu
from jax.experimental.pallas import tpu_sc as plsc

N = 10000
E = 320000
D = 128

NC = 2
NS = 16
NW = NC * NS

NPAD = 10240
CH = 128
NCHUNK = 80
EPW = CH * NCHUNK
EPAD = NW * EPW
RPS = NPAD // NS
DUMP = NPAD - 1


IB = 40
NBLK = NCHUNK // IB


def _sc_agg_body(h_hbm, src3_hbm, dst3_hbm, z128_hbm, agg_out,
                 sidx_blk, didx_blk, rows0, rows1,
                 gsem0, gsem1, ssem0, ssem1, acc):
  c = lax.axis_index("c")
  s = lax.axis_index("s")
  wid = c * NS + s
  r0 = s * RPS
  rows = (rows0, rows1)
  gsem = (gsem0, gsem1)
  ssem = (ssem0, ssem1)

  pltpu.sync_copy(z128_hbm.at[pl.ds(r0, RPS), :], acc.at[pl.ds(r0, RPS), :])
  plsc.subcore_barrier()

  def gather_start(j, b):
    pltpu.async_copy(h_hbm.at[sidx_blk.at[j]], rows[b], gsem[b])

  def gather_wait(j, b):
    pltpu.make_async_copy(h_hbm.at[sidx_blk.at[j]], rows[b], gsem[b]).wait()

  def scat_start(j, b):
    pltpu.async_copy(rows[b], acc.at[didx_blk.at[j]], ssem[b], add=True)

  def scat_wait(j, b):
    pltpu.make_async_copy(rows[b], acc.at[didx_blk.at[j]], ssem[b]).wait()

  @pl.loop(0, NBLK)
  def _(blk):
    pltpu.sync_copy(src3_hbm.at[wid, pl.ds(blk * IB, IB)], sidx_blk)
    pltpu.sync_copy(dst3_hbm.at[wid, pl.ds(blk * IB, IB)], didx_blk)
    gather_start(0, 0)

    @pl.loop(0, IB // 2)
    def _(p):
      for b in range(2):
        j = 2 * p + b
        gather_wait(j, b)
        scat_start(j, b)

        @pl.when(j >= 1)
        def _():
          scat_wait(j - 1, 1 - b)

        @pl.when(j + 1 < IB)
        def _():
          gather_start(j + 1, 1 - b)

    scat_wait(IB - 1, 1)

  plsc.subcore_barrier()
  pltpu.sync_copy(acc.at[pl.ds(r0, RPS), :], agg_out.at[c, pl.ds(r0, RPS), :])


_sc_agg = pl.kernel(
    _sc_agg_body,
    out_type=jax.ShapeDtypeStruct((NC, NPAD, D), jnp.float32),
    mesh=plsc.VectorSubcoreMesh(core_axis_name="c", subcore_axis_name="s",
                                num_cores=NC, num_subcores=NS),
    scratch_types=[
        pltpu.VMEM((IB, CH), jnp.int32),
        pltpu.VMEM((IB, CH), jnp.int32),
        pltpu.VMEM((CH, D), jnp.float32),
        pltpu.VMEM((CH, D), jnp.float32),
        pltpu.SemaphoreType.DMA,
        pltpu.SemaphoreType.DMA,
        pltpu.SemaphoreType.DMA,
        pltpu.SemaphoreType.DMA,
        pltpu.VMEM_SHARED((NPAD, D), jnp.float32),
    ],
)


def _sc_count_body(dst3_hbm, ones_hbm, z128_hbm, cnt_out, didx_all, ones,
                   ssem, acc):
  c = lax.axis_index("c")
  s = lax.axis_index("s")
  wid = c * NS + s
  r0 = s * RPS

  pltpu.sync_copy(dst3_hbm.at[wid], didx_all)
  pltpu.sync_copy(z128_hbm.at[pl.ds(r0, RPS), :], acc.at[pl.ds(r0, RPS), :])
  pltpu.sync_copy(ones_hbm, ones)
  plsc.subcore_barrier()

  @pl.loop(0, NCHUNK)
  def _(i):
    pltpu.async_copy(ones, acc.at[didx_all.at[i]], ssem, add=True)

    @pl.when(i > 0)
    def _():
      pltpu.make_async_copy(ones, acc.at[didx_all.at[i]], ssem).wait()

  pltpu.make_async_copy(ones, acc.at[didx_all.at[0]], ssem).wait()
  plsc.subcore_barrier()
  pltpu.sync_copy(acc.at[pl.ds(r0, RPS), :], cnt_out.at[c, pl.ds(r0, RPS), :])


_sc_count = pl.kernel(
    _sc_count_body,
    out_type=jax.ShapeDtypeStruct((NC, NPAD, D), jnp.float32),
    mesh=plsc.VectorSubcoreMesh(core_axis_name="c", subcore_axis_name="s",
                                num_cores=NC, num_subcores=NS),
    scratch_types=[
        pltpu.VMEM((NCHUNK, CH), jnp.int32),
        pltpu.VMEM((CH, D), jnp.float32),
        pltpu.SemaphoreType.DMA,
        pltpu.VMEM_SHARED((NPAD, D), jnp.float32),
    ],
)

RB = 2048


def _tc_layer_body(relu, agg_ref, cnt_ref, h_ref, wl_ref, bl_ref, wr_ref,
                   out_ref):
  a = agg_ref[0] + agg_ref[1]
  cnt = cnt_ref[0, :, :8] + cnt_ref[1, :, :8]
  inv = 1.0 / jnp.maximum(cnt[:, :1], 1.0)
  m = a * inv
  dn = (((1,), (1,)), ((), ()))
  y = (lax.dot_general(m, wl_ref[...], dn, preferred_element_type=jnp.float32)
       + lax.dot_general(h_ref[...], wr_ref[...], dn,
                         preferred_element_type=jnp.float32)
       + bl_ref[...])
  out_ref[...] = jnp.maximum(y, 0.0) if relu else y


def _tc_layer(relu, agg2, cnt2, h, Wl, bl, Wr):
  return pl.pallas_call(
      functools.partial(_tc_layer_body, relu),
      grid=(NPAD // RB,),
      in_specs=[
          pl.BlockSpec((NC, RB, D), lambda i: (0, i, 0)),
          pl.BlockSpec((NC, RB, D), lambda i: (0, i, 0)),
          pl.BlockSpec((RB, D), lambda i: (i, 0)),
          pl.BlockSpec((D, D), lambda i: (0, 0)),
          pl.BlockSpec((1, D), lambda i: (0, 0)),
          pl.BlockSpec((D, D), lambda i: (0, 0)),
      ],
      out_specs=pl.BlockSpec((RB, D), lambda i: (i, 0)),
      out_shape=jax.ShapeDtypeStruct((NPAD, D), jnp.float32),
  )(agg2, cnt2, h, Wl, bl.reshape(1, D), Wr)


def kernel(x, edge_index, Wl0, bl0, Wr0, Wl1, bl1, Wr1, Wl2, bl2, Wr2):
  epad = EPAD - E
  src = jnp.concatenate(
      [edge_index[0].astype(jnp.int32),
       jnp.arange(epad, dtype=jnp.int32) % N])
  dst = jnp.concatenate(
      [edge_index[1].astype(jnp.int32), jnp.full((epad,), DUMP, jnp.int32)])
  src = src.reshape(NW, NCHUNK, CH)
  dst = dst.reshape(NW, NCHUNK, CH)
  h = jnp.pad(x, ((0, NPAD - N), (0, 0)))
  z128 = jnp.zeros((NPAD, D), jnp.float32)
  ones128 = jnp.ones((CH, D), jnp.float32)

  cnt2 = _sc_count(dst, ones128, z128)
  agg2 = _sc_agg(h, src, dst, z128)
  h = _tc_layer(True, agg2, cnt2, h, Wl0, bl0, Wr0)
  agg2 = _sc_agg(h, src, dst, z128)
  h = _tc_layer(True, agg2, cnt2, h, Wl1, bl1, Wr1)
  agg2 = _sc_agg(h, src, dst, z128)
  return _tc_layer(False, agg2, cnt2, h, Wl2, bl2, Wr2)[:N]
